# baseline (speedup 1.0000x reference)
import jax
import jax.numpy as jnp
from jax.experimental import pallas as pl

D = 128
_BLK = 5000


def _proj_body(x_ref, w_ref, a_ref, h_ref, al_ref):
    x = x_ref[...]
    h = jnp.dot(x, w_ref[...], preferred_element_type=jnp.float32)
    h_ref[...] = h
    al_ref[...] = jnp.sum(h * a_ref[...], axis=1, keepdims=True)


def _proj(x, w, a):
    n = x.shape[0]
    grid = (n // _BLK,)
    h, al = pl.pallas_call(
        _proj_body,
        grid=grid,
        in_specs=[
            pl.BlockSpec((_BLK, D), lambda i: (i, 0)),
            pl.BlockSpec((D, D), lambda i: (0, 0)),
            pl.BlockSpec((1, D), lambda i: (0, 0)),
        ],
        out_specs=[
            pl.BlockSpec((_BLK, D), lambda i: (i, 0)),
            pl.BlockSpec((_BLK, 1), lambda i: (i, 0)),
        ],
        out_shape=[
            jax.ShapeDtypeStruct((n, D), jnp.float32),
            jax.ShapeDtypeStruct((n, 1), jnp.float32),
        ],
    )(x, w.astype(jnp.float32), a.reshape(1, D).astype(jnp.float32))
    return h, al[:, 0]


def _combine2_body(o1_ref, o2_ref, b_ref, out_ref):
    out_ref[...] = jnp.maximum(
        (o1_ref[...] + o2_ref[...]) * 0.5 + b_ref[...], 0.0)


def _combine2(o1, o2, b):
    n = o1.shape[0]
    return pl.pallas_call(
        _combine2_body,
        grid=(n // _BLK,),
        in_specs=[
            pl.BlockSpec((_BLK, D), lambda i: (i, 0)),
            pl.BlockSpec((_BLK, D), lambda i: (i, 0)),
            pl.BlockSpec((1, D), lambda i: (0, 0)),
        ],
        out_specs=pl.BlockSpec((_BLK, D), lambda i: (i, 0)),
        out_shape=jax.ShapeDtypeStruct((n, D), jnp.float32),
    )(o1, o2, b.reshape(1, D))


def _combine1_body(o_ref, b_ref, out_ref):
    out_ref[...] = jnp.maximum(o_ref[...] + b_ref[...], 0.0)


def _combine1(o, b):
    n = o.shape[0]
    return pl.pallas_call(
        _combine1_body,
        grid=(n // _BLK,),
        in_specs=[
            pl.BlockSpec((_BLK, D), lambda i: (i, 0)),
            pl.BlockSpec((1, D), lambda i: (0, 0)),
        ],
        out_specs=pl.BlockSpec((_BLK, D), lambda i: (i, 0)),
        out_shape=jax.ShapeDtypeStruct((n, D), jnp.float32),
    )(o, b.reshape(1, D))


def _final_body(x_ref, w_ref, b_ref, o_ref):
    o_ref[...] = jnp.dot(
        x_ref[...], w_ref[...], preferred_element_type=jnp.float32) + b_ref[...]


def _final(x, w, b):
    n = x.shape[0]
    return pl.pallas_call(
        _final_body,
        grid=(n // _BLK,),
        in_specs=[
            pl.BlockSpec((_BLK, D), lambda i: (i, 0)),
            pl.BlockSpec((D, D), lambda i: (0, 0)),
            pl.BlockSpec((1, D), lambda i: (0, 0)),
        ],
        out_specs=pl.BlockSpec((_BLK, D), lambda i: (i, 0)),
        out_shape=jax.ShapeDtypeStruct((n, D), jnp.float32),
    )(x, w, b.reshape(1, D))


def _gat_aggregate(hs, als, ald, ei, n_dst):
    src, dst = ei[0], ei[1]
    logit = als[src] + ald[dst]
    logit = jnp.where(logit >= 0.0, logit, 0.2 * logit)
    m = jax.ops.segment_max(logit, dst, num_segments=n_dst)
    m = jnp.where(jnp.isfinite(m), m, 0.0)
    e = jnp.exp(logit - m[dst])
    denom = jax.ops.segment_sum(e, dst, num_segments=n_dst) + 1e-16
    coef = e / denom[dst]
    return jax.ops.segment_sum(coef[:, None] * hs[src], dst,
                               num_segments=n_dst)


def kernel(x_patient, x_drug, x_effect, ei_takes, ei_rev_takes,
           ei_experiences, ei_rev_experiences, Wsrc, Wdst, att_src, att_dst,
           bias_rel, W_lin, b_lin):
    xd = {'patient': x_patient, 'drug': x_drug, 'effect': x_effect}
    rels = [('patient', 'drug', ei_takes),
            ('drug', 'patient', ei_rev_takes),
            ('patient', 'effect', ei_experiences),
            ('effect', 'patient', ei_rev_experiences)]

    o = [None] * 4
    for r, (st, dt, ei) in enumerate(rels):
        hs, als = _proj(xd[st], Wsrc[0, r], att_src[0, r])
        _, ald = _proj(xd[dt], Wdst[0, r], att_dst[0, r])
        o[r] = _gat_aggregate(hs, als, ald, ei, xd[dt].shape[0])
    xd = {
        'drug': _combine1(o[0], bias_rel[0, 0]),
        'patient': _combine2(o[1], o[3],
                             (bias_rel[0, 1] + bias_rel[0, 3]) * 0.5),
        'effect': _combine1(o[2], bias_rel[0, 2]),
    }

    hs1, als1 = _proj(xd['drug'], Wsrc[1, 1], att_src[1, 1])
    _, ald1 = _proj(xd['patient'], Wdst[1, 1], att_dst[1, 1])
    o1 = _gat_aggregate(hs1, als1, ald1, ei_rev_takes, xd['patient'].shape[0])

    hs3, als3 = _proj(xd['effect'], Wsrc[1, 3], att_src[1, 3])
    _, ald3 = _proj(xd['patient'], Wdst[1, 3], att_dst[1, 3])
    o3 = _gat_aggregate(hs3, als3, ald3, ei_rev_experiences,
                        xd['patient'].shape[0])

    xp = _combine2(o1, o3, (bias_rel[1, 1] + bias_rel[1, 3]) * 0.5)
    return _final(xp, W_lin, b_lin)


# speedup vs baseline: 1.3002x; 1.3002x over previous
import jax
import jax.numpy as jnp
from jax.experimental import pallas as pl

D = 128
_BLK = 5000


def _proj_body(x_ref, w_ref, a_ref, h_ref, al_ref):
    x = x_ref[...]
    h = jnp.dot(x, w_ref[...], preferred_element_type=jnp.float32)
    h_ref[...] = h
    al_ref[...] = jnp.sum(h * a_ref[...], axis=1, keepdims=True)


def _proj(x, w, a):
    n = x.shape[0]
    grid = (n // _BLK,)
    h, al = pl.pallas_call(
        _proj_body,
        grid=grid,
        in_specs=[
            pl.BlockSpec((_BLK, D), lambda i: (i, 0)),
            pl.BlockSpec((D, D), lambda i: (0, 0)),
            pl.BlockSpec((1, D), lambda i: (0, 0)),
        ],
        out_specs=[
            pl.BlockSpec((_BLK, D), lambda i: (i, 0)),
            pl.BlockSpec((_BLK, 1), lambda i: (i, 0)),
        ],
        out_shape=[
            jax.ShapeDtypeStruct((n, D), jnp.float32),
            jax.ShapeDtypeStruct((n, 1), jnp.float32),
        ],
    )(x, w.astype(jnp.float32), a.reshape(1, D).astype(jnp.float32))
    return h, al[:, 0]


def _combine2_body(o1_ref, d1_ref, o2_ref, d2_ref, b_ref, out_ref):
    out_ref[...] = jnp.maximum(
        (o1_ref[...] / d1_ref[...] + o2_ref[...] / d2_ref[...]) * 0.5
        + b_ref[...], 0.0)


def _combine2(o1, d1, o2, d2, b):
    n = o1.shape[0]
    return pl.pallas_call(
        _combine2_body,
        grid=(n // _BLK,),
        in_specs=[
            pl.BlockSpec((_BLK, D), lambda i: (i, 0)),
            pl.BlockSpec((_BLK, 1), lambda i: (i, 0)),
            pl.BlockSpec((_BLK, D), lambda i: (i, 0)),
            pl.BlockSpec((_BLK, 1), lambda i: (i, 0)),
            pl.BlockSpec((1, D), lambda i: (0, 0)),
        ],
        out_specs=pl.BlockSpec((_BLK, D), lambda i: (i, 0)),
        out_shape=jax.ShapeDtypeStruct((n, D), jnp.float32),
    )(o1, d1.reshape(n, 1), o2, d2.reshape(n, 1), b.reshape(1, D))


def _combine1_body(o_ref, d_ref, b_ref, out_ref):
    out_ref[...] = jnp.maximum(o_ref[...] / d_ref[...] + b_ref[...], 0.0)


def _combine1(o, d, b):
    n = o.shape[0]
    return pl.pallas_call(
        _combine1_body,
        grid=(n // _BLK,),
        in_specs=[
            pl.BlockSpec((_BLK, D), lambda i: (i, 0)),
            pl.BlockSpec((_BLK, 1), lambda i: (i, 0)),
            pl.BlockSpec((1, D), lambda i: (0, 0)),
        ],
        out_specs=pl.BlockSpec((_BLK, D), lambda i: (i, 0)),
        out_shape=jax.ShapeDtypeStruct((n, D), jnp.float32),
    )(o, d.reshape(n, 1), b.reshape(1, D))


def _final_body(x_ref, w_ref, b_ref, o_ref):
    o_ref[...] = jnp.dot(
        x_ref[...], w_ref[...], preferred_element_type=jnp.float32) + b_ref[...]


def _final(x, w, b):
    n = x.shape[0]
    return pl.pallas_call(
        _final_body,
        grid=(n // _BLK,),
        in_specs=[
            pl.BlockSpec((_BLK, D), lambda i: (i, 0)),
            pl.BlockSpec((D, D), lambda i: (0, 0)),
            pl.BlockSpec((1, D), lambda i: (0, 0)),
        ],
        out_specs=pl.BlockSpec((_BLK, D), lambda i: (i, 0)),
        out_shape=jax.ShapeDtypeStruct((n, D), jnp.float32),
    )(x, w, b.reshape(1, D))


def _sort_rel(ei):
    order = jnp.argsort(ei[1])
    return ei[0][order], ei[1][order]


def _gat_parts(hs, als, ald, src, dst, n_dst):
    logit = als[src] + ald[dst]
    logit = jnp.where(logit >= 0.0, logit, 0.2 * logit)
    m = jax.ops.segment_max(logit, dst, num_segments=n_dst,
                            indices_are_sorted=True)
    m = jnp.where(jnp.isfinite(m), m, 0.0)
    e = jnp.exp(logit - m[dst])
    denom = jax.ops.segment_sum(e, dst, num_segments=n_dst,
                                indices_are_sorted=True) + 1e-16
    num = jax.ops.segment_sum(e[:, None] * hs[src], dst, num_segments=n_dst,
                              indices_are_sorted=True)
    return num, denom


def kernel(x_patient, x_drug, x_effect, ei_takes, ei_rev_takes,
           ei_experiences, ei_rev_experiences, Wsrc, Wdst, att_src, att_dst,
           bias_rel, W_lin, b_lin):
    xd = {'patient': x_patient, 'drug': x_drug, 'effect': x_effect}
    rels = [('patient', 'drug', _sort_rel(ei_takes)),
            ('drug', 'patient', _sort_rel(ei_rev_takes)),
            ('patient', 'effect', _sort_rel(ei_experiences)),
            ('effect', 'patient', _sort_rel(ei_rev_experiences))]

    o = [None] * 4
    d = [None] * 4
    for r, (st, dt, (src, dst)) in enumerate(rels):
        hs, als = _proj(xd[st], Wsrc[0, r], att_src[0, r])
        _, ald = _proj(xd[dt], Wdst[0, r], att_dst[0, r])
        o[r], d[r] = _gat_parts(hs, als, ald, src, dst, xd[dt].shape[0])
    xd = {
        'drug': _combine1(o[0], d[0], bias_rel[0, 0]),
        'patient': _combine2(o[1], d[1], o[3], d[3],
                             (bias_rel[0, 1] + bias_rel[0, 3]) * 0.5),
        'effect': _combine1(o[2], d[2], bias_rel[0, 2]),
    }

    src1, dst1 = rels[1][2]
    hs1, als1 = _proj(xd['drug'], Wsrc[1, 1], att_src[1, 1])
    _, ald1 = _proj(xd['patient'], Wdst[1, 1], att_dst[1, 1])
    o1, d1 = _gat_parts(hs1, als1, ald1, src1, dst1, xd['patient'].shape[0])

    src3, dst3 = rels[3][2]
    hs3, als3 = _proj(xd['effect'], Wsrc[1, 3], att_src[1, 3])
    _, ald3 = _proj(xd['patient'], Wdst[1, 3], att_dst[1, 3])
    o3, d3 = _gat_parts(hs3, als3, ald3, src3, dst3, xd['patient'].shape[0])

    xp = _combine2(o1, d1, o3, d3, (bias_rel[1, 1] + bias_rel[1, 3]) * 0.5)
    return _final(xp, W_lin, b_lin)
